# baseline (speedup 1.0000x reference)
import jax
import jax.numpy as jnp
from jax.experimental import pallas as pl
from jax.experimental.pallas import tpu as pltpu


def _gcn_kernel(x_ref, adje_ref, adjo_ref, w_ref, b_ref, o_ref, y_ref):
    S, B, H = x_ref.shape
    O = w_ref.shape[1]
    j = pl.program_id(0)

    @pl.when(j == 0)
    def _():
        xf = x_ref[...].reshape(S * B, H).astype(jnp.bfloat16)
        y = jnp.dot(xf, w_ref[...],
                    preferred_element_type=jnp.float32) + b_ref[...]
        y3 = y.astype(jnp.bfloat16).reshape(S, B, O)
        for bb in range(B):
            y_ref[bb * S:(bb + 1) * S, :] = y3[:, bb, :]

    be = 2 * j
    ye = y_ref[pl.ds(be * S, S), :]
    ze = jnp.dot(adje_ref[...], ye, preferred_element_type=jnp.float32)
    o_ref[:, be, :] = jnp.maximum(ze, 0.0)

    bo = 2 * j + 1
    yo = y_ref[pl.ds(bo * S, S), :]
    zo = jnp.dot(adjo_ref[...], yo, preferred_element_type=jnp.float32)
    o_ref[:, bo, :] = jnp.maximum(zo, 0.0)


def kernel(x, adj, w, b):
    S, B, H = x.shape
    O = w.shape[0]

    adj_bm = jnp.transpose(adj.astype(jnp.bfloat16), (2, 0, 1))
    adj_e = adj_bm[0::2]
    adj_o = adj_bm[1::2]
    w_t = jnp.transpose(w).astype(jnp.bfloat16)
    b2d = b.reshape(1, O).astype(jnp.float32)

    return pl.pallas_call(
        _gcn_kernel,
        out_shape=jax.ShapeDtypeStruct((S, B, O), jnp.float32),
        grid_spec=pltpu.PrefetchScalarGridSpec(
            num_scalar_prefetch=0,
            grid=(B // 2,),
            in_specs=[
                pl.BlockSpec((S, B, H), lambda j: (0, 0, 0)),
                pl.BlockSpec((None, S, S), lambda j: (j, 0, 0)),
                pl.BlockSpec((None, S, S), lambda j: (j, 0, 0)),
                pl.BlockSpec((H, O), lambda j: (0, 0)),
                pl.BlockSpec((1, O), lambda j: (0, 0)),
            ],
            out_specs=pl.BlockSpec((S, B, O), lambda j: (0, 0, 0)),
            scratch_shapes=[pltpu.VMEM((B * S, O), jnp.bfloat16)],
        ),
        compiler_params=pltpu.CompilerParams(
            dimension_semantics=("arbitrary",),
            vmem_limit_bytes=64 * 1024 * 1024,
        ),
    )(x, adj_e, adj_o, w_t, b2d)


# speedup vs baseline: 1.1367x; 1.1367x over previous
import jax
import jax.numpy as jnp
from jax.experimental import pallas as pl
from jax.experimental.pallas import tpu as pltpu


def _gcn_kernel(x_ref, adj_ref, w_ref, b_ref, o_ref, y_ref):
    S, B, H = x_ref.shape
    O = w_ref.shape[0]
    j = pl.program_id(0)

    @pl.when(j == 0)
    def _():
        xf = x_ref[...].reshape(S * B, H).astype(jnp.bfloat16)
        y = jax.lax.dot_general(
            xf, w_ref[...].astype(jnp.bfloat16),
            dimension_numbers=(((1,), (1,)), ((), ())),
            preferred_element_type=jnp.float32) + b_ref[...]
        y3 = y.astype(jnp.bfloat16).reshape(S, B, O)
        for bb in range(B):
            y_ref[bb * S:(bb + 1) * S, :] = y3[:, bb, :]

    y_b = y_ref[pl.ds(j * S, S), :]
    z = jnp.dot(adj_ref[...], y_b,
                preferred_element_type=jnp.float32)
    o_ref[:, j, :] = jnp.maximum(z, 0.0)


def kernel(x, adj, w, b):
    S, B, H = x.shape
    O = w.shape[0]

    adj_bm = jnp.transpose(adj.astype(jnp.bfloat16), (2, 0, 1))
    b2d = b.reshape(1, O).astype(jnp.float32)

    return pl.pallas_call(
        _gcn_kernel,
        out_shape=jax.ShapeDtypeStruct((S, B, O), jnp.float32),
        grid_spec=pltpu.PrefetchScalarGridSpec(
            num_scalar_prefetch=0,
            grid=(B,),
            in_specs=[
                pl.BlockSpec((S, B, H), lambda j: (0, 0, 0)),
                pl.BlockSpec((None, S, S), lambda j: (j, 0, 0)),
                pl.BlockSpec((O, H), lambda j: (0, 0)),
                pl.BlockSpec((1, O), lambda j: (0, 0)),
            ],
            out_specs=pl.BlockSpec((S, B, O), lambda j: (0, 0, 0)),
            scratch_shapes=[pltpu.VMEM((B * S, O), jnp.bfloat16)],
        ),
        compiler_params=pltpu.CompilerParams(
            dimension_semantics=("arbitrary",),
            vmem_limit_bytes=64 * 1024 * 1024,
        ),
    )(x, adj_bm, w, b2d)
